# baseline (speedup 1.0000x reference)
import numpy as np
import jax
import jax.numpy as jnp
from jax import lax
from jax.experimental import pallas as pl
from jax.experimental.pallas import tpu as pltpu

_H = _W = 512
_B = 32
_NPI = 512
_IMGS = 8

_gen = np.random.default_rng(0)
_RX = int(_gen.integers(0, _W - 3 + 1))
_RY = int(_gen.integers(0, _H - 3 + 1))

_PAT = (((np.arange(_NPI)[:, None] >> np.arange(8, -1, -1)[None, :]) & 1)
        .astype(np.float32).T.copy())

_NOISE = np.asarray(
    jax.random.uniform(jax.random.key(42), (_B, _NPI), jnp.float32)) * 0.5


def _fused_body(w_ref, t_ref, p_ref, n_ref, x_ref, o_ref, bits_scr):
    step = pl.program_id(0)

    @pl.when(step == 0)
    def _search():
        err = jnp.zeros((_B, _NPI), jnp.float32)
        for i in range(1, 6):
            for j in range(1, 6):
                ws = None
                ps = None
                for a in (i - 1, i, i + 1):
                    for b in (j - 1, j, j + 1):
                        if 2 <= a <= 4 and 2 <= b <= 4:
                            k = 3 * (a - 2) + (b - 2)
                            v = p_ref[k:k + 1, :]
                            ps = v if ps is None else ps + v
                        else:
                            v = w_ref[:, 7 * a + b: 7 * a + b + 1]
                            ws = v if ws is None else ws + v
                ssum = ps if ws is None else (ws + ps)
                if 2 <= i <= 4 and 2 <= j <= 4:
                    c = p_ref[3 * (i - 2) + (j - 2)
                              : 3 * (i - 2) + (j - 2) + 1, :]
                else:
                    c = w_ref[:, 7 * i + j: 7 * i + j + 1]
                e = (jnp.clip(ssum - 2.0, 0.0, 1.0)
                     - jnp.clip(ssum - c - 3.0, 0.0, 1.0))
                t = t_ref[:, 5 * (i - 1) + (j - 1): 5 * (i - 1) + (j - 1) + 1]
                err = err + jnp.abs(e - t)
        seeded = err + n_ref[...]
        m = jnp.min(seeded, axis=1, keepdims=True)
        ji = lax.broadcasted_iota(jnp.int32, (_B, _NPI), 1)
        idx = jnp.min(jnp.where(seeded == m, ji, _NPI), axis=1, keepdims=True)
        kk = lax.broadcasted_iota(jnp.int32, (_B, 9), 1)
        bits_scr[...] = ((idx >> (8 - kk)) & 1).astype(jnp.float32)

    o_ref[...] = x_ref[...]
    for s in range(_B // _IMGS):
        @pl.when(step == s)
        def _patch(s=s):
            for m in range(_IMGS):
                row = bits_scr[s * _IMGS + m, :]
                for i in range(3):
                    o_ref[m, 0, _RY + i, _RX:_RX + 3] = row[3 * i:3 * i + 3]


def kernel(x, target):
    B = x.shape[0]
    wins = lax.slice(x, (0, 0, _RY - 2, _RX - 2),
                     (B, 1, _RY + 5, _RX + 5)).reshape(B, 49)
    tws = lax.slice(target, (0, 0, _RY - 1, _RX - 1),
                    (B, 1, _RY + 4, _RX + 4)).reshape(B, 25)
    out = pl.pallas_call(
        _fused_body,
        grid=(B // _IMGS,),
        in_specs=[
            pl.BlockSpec((_B, 49), lambda b: (0, 0)),
            pl.BlockSpec((_B, 25), lambda b: (0, 0)),
            pl.BlockSpec((9, _NPI), lambda b: (0, 0)),
            pl.BlockSpec((_B, _NPI), lambda b: (0, 0)),
            pl.BlockSpec((_IMGS, 1, _H, _W), lambda b: (b, 0, 0, 0)),
        ],
        out_specs=pl.BlockSpec((_IMGS, 1, _H, _W), lambda b: (b, 0, 0, 0)),
        out_shape=jax.ShapeDtypeStruct(x.shape, x.dtype),
        scratch_shapes=[pltpu.VMEM((_B, 9), jnp.float32)],
        compiler_params=pltpu.CompilerParams(
            dimension_semantics=("arbitrary",)),
    )(wins, tws, jnp.asarray(_PAT), jnp.asarray(_NOISE), x)
    return out
